# baseline (speedup 1.0000x reference)
import functools

import jax
import jax.numpy as jnp
from jax import lax
from jax.experimental import pallas as pl
from jax.experimental.pallas import tpu as pltpu
from jax.experimental.pallas import tpu_sc as plsc

NC, NS, L = 2, 16, 16
NW = NS


def _body(dim, bw,
          t_tab, s_tab, tweet_r, sent_r, out_r, ttab_v, stab_v, tidx, sidx,
          big, sem):
    wid = lax.axis_index("s")
    cps = [
        pltpu.async_copy(t_tab, ttab_v, sem),
        pltpu.async_copy(s_tab, stab_v, sem),
        pltpu.async_copy(tweet_r.at[wid], tidx, sem),
        pltpu.async_copy(sent_r.at[wid], sidx, sem),
    ]
    for c in cps:
        c.wait()

    lanes = lax.iota(jnp.int32, L)

    def chunk(ch, _):
        base = pl.multiple_of(ch * L, L)
        t16 = tidx[pl.ds(base, L)]
        s16 = sidx[pl.ds(base, L)]
        m = t16 == 0
        trow = t16 * dim
        srow = s16 * dim
        obase = (lanes + base) * (2 * dim)
        zero = jnp.zeros((L,), jnp.float32)
        tv = [plsc.load_gather(ttab_v, [trow + c]) for c in range(dim)]
        tv = [jnp.where(m, zero, v) for v in tv]
        for c in range(dim):
            plsc.store_scatter(big, [obase + c], tv[c])
        sv = [plsc.load_gather(stab_v, [srow + c]) for c in range(dim)]
        sv = [jnp.where(m, zero, v) for v in sv]
        for c in range(dim):
            plsc.store_scatter(big, [obase + (dim + c)], sv[c])
        return _

    lax.fori_loop(0, bw // L, chunk, None)
    pltpu.sync_copy(big, out_r.at[wid])


def kernel(tweet, sentiment, tweet_table, sentiment_table):
    b = tweet.shape[0]
    dim = tweet_table.shape[1]
    bw = b // NW

    t_flat = tweet_table.reshape(-1)
    s_flat = sentiment_table.reshape(-1)
    tweet_r = tweet.astype(jnp.int32).reshape(NW, bw)
    sent_r = sentiment.astype(jnp.int32).reshape(NW, bw)

    mesh = plsc.VectorSubcoreMesh(core_axis_name="c", subcore_axis_name="s", num_cores=1)
    run = pl.kernel(
        functools.partial(_body, dim, bw),
        out_type=jax.ShapeDtypeStruct((NW, bw * 2 * dim), jnp.float32),
        mesh=mesh,
        scratch_types=[
            pltpu.VMEM((t_flat.shape[0],), jnp.float32),
            pltpu.VMEM((s_flat.shape[0],), jnp.float32),
            pltpu.VMEM((bw,), jnp.int32),
            pltpu.VMEM((bw,), jnp.int32),
            pltpu.VMEM((bw * 2 * dim,), jnp.float32),
            pltpu.SemaphoreType.DMA,
        ],
        compiler_params=pltpu.CompilerParams(needs_layout_passes=False),
    )
    out = run(t_flat, s_flat, tweet_r, sent_r)
    return out.reshape(b, 2 * dim)


# speedup vs baseline: 1.3778x; 1.3778x over previous
import functools

import jax
import jax.numpy as jnp
from jax import lax
from jax.experimental import pallas as pl
from jax.experimental.pallas import tpu as pltpu
from jax.experimental.pallas import tpu_sc as plsc

NC, NS, L = 2, 16, 16
NW = NC * NS


def _body(dim, bw,
          t_tab, s_tab, tweet_r, sent_r, out_r, ttab_v, stab_v, tidx, sidx,
          big, sem):
    wid = lax.axis_index("s") * NC + lax.axis_index("c")
    with jax.named_scope("prologue_dma"):
        cps = [
            pltpu.async_copy(t_tab, ttab_v, sem),
            pltpu.async_copy(s_tab, stab_v, sem),
            pltpu.async_copy(tweet_r.at[wid], tidx, sem),
            pltpu.async_copy(sent_r.at[wid], sidx, sem),
        ]
        for c in cps:
            c.wait()

    lanes = lax.iota(jnp.int32, L)

    def chunk(ch, _):
        base = pl.multiple_of(ch * L, L)
        t16 = tidx[pl.ds(base, L)]
        s16 = sidx[pl.ds(base, L)]
        m = t16 == 0
        trow = t16 * dim
        srow = s16 * dim
        obase = (lanes + base) * (2 * dim)
        zero = jnp.zeros((L,), jnp.float32)
        tv = [plsc.load_gather(ttab_v, [trow + c]) for c in range(dim)]
        tv = [jnp.where(m, zero, v) for v in tv]
        for c in range(dim):
            plsc.store_scatter(big, [obase + c], tv[c])
        sv = [plsc.load_gather(stab_v, [srow + c]) for c in range(dim)]
        sv = [jnp.where(m, zero, v) for v in sv]
        for c in range(dim):
            plsc.store_scatter(big, [obase + (dim + c)], sv[c])
        return _

    with jax.named_scope("gather_loop"):
        lax.fori_loop(0, bw // L, chunk, None)
    with jax.named_scope("out_dma"):
        pltpu.sync_copy(big, out_r.at[wid])


def kernel(tweet, sentiment, tweet_table, sentiment_table):
    b = tweet.shape[0]
    dim = tweet_table.shape[1]
    bw = b // NW

    t_flat = tweet_table.reshape(-1)
    s_flat = sentiment_table.reshape(-1)
    tweet_r = tweet.astype(jnp.int32).reshape(NW, bw)
    sent_r = sentiment.astype(jnp.int32).reshape(NW, bw)

    mesh = plsc.VectorSubcoreMesh(core_axis_name="c", subcore_axis_name="s")
    run = pl.kernel(
        functools.partial(_body, dim, bw),
        out_type=jax.ShapeDtypeStruct((NW, bw * 2 * dim), jnp.float32),
        mesh=mesh,
        scratch_types=[
            pltpu.VMEM((t_flat.shape[0],), jnp.float32),
            pltpu.VMEM((s_flat.shape[0],), jnp.float32),
            pltpu.VMEM((bw,), jnp.int32),
            pltpu.VMEM((bw,), jnp.int32),
            pltpu.VMEM((bw * 2 * dim,), jnp.float32),
            pltpu.SemaphoreType.DMA,
        ],
        compiler_params=pltpu.CompilerParams(needs_layout_passes=False),
    )
    out = run(t_flat, s_flat, tweet_r, sent_r)
    return out.reshape(b, 2 * dim)


# speedup vs baseline: 2.0792x; 1.5091x over previous
import functools

import jax
import jax.numpy as jnp
from jax import lax
from jax.experimental import pallas as pl
from jax.experimental.pallas import tpu as pltpu
from jax.experimental.pallas import tpu_sc as plsc

NC, NS, L = 2, 16, 16
NW = NC * NS


def _body(dim, tz, sz, bw,
          t_tab, s_tab, tweet_r, sent_r, out_r, ttab_v, stab_v, tidx, sidx,
          big, sem):
    wid = lax.axis_index("s") * NC + lax.axis_index("c")
    cps = [
        pltpu.async_copy(t_tab, ttab_v, sem),
        pltpu.async_copy(s_tab, stab_v, sem),
        pltpu.async_copy(tweet_r.at[wid], tidx, sem),
        pltpu.async_copy(sent_r.at[wid], sidx, sem),
    ]
    for c in cps:
        c.wait()

    lanes = lax.iota(jnp.int32, L)

    def chunk(ch, _):
        base = pl.multiple_of(ch * L, L)
        t16 = tidx[pl.ds(base, L)]
        s16 = sidx[pl.ds(base, L)]
        m = t16 == 0
        trow = jnp.where(m, tz, t16) * dim
        srow = jnp.where(m, sz, s16) * dim
        obase_t = (lanes + base) * (2 * dim)
        obase_s = obase_t + dim
        for g in range(0, dim, 16):
            cols = [(lanes + (g + k)) & (dim - 1) for k in range(16)]
            tv = [plsc.load_gather(ttab_v, [trow + cl]) for cl in cols]
            for k, cl in enumerate(cols):
                plsc.store_scatter(big, [obase_t + cl], tv[k])
            sv = [plsc.load_gather(stab_v, [srow + cl]) for cl in cols]
            for k, cl in enumerate(cols):
                plsc.store_scatter(big, [obase_s + cl], sv[k])
        return _

    lax.fori_loop(0, bw // L, chunk, None)
    pltpu.sync_copy(big, out_r.at[wid])


def kernel(tweet, sentiment, tweet_table, sentiment_table):
    b = tweet.shape[0]
    dim = tweet_table.shape[1]
    tz = tweet_table.shape[0]
    sz = sentiment_table.shape[0]
    bw = b // NW

    zrow = jnp.zeros((1, dim), jnp.float32)
    t_flat = jnp.concatenate([tweet_table, zrow], axis=0).reshape(-1)
    s_flat = jnp.concatenate([sentiment_table, zrow], axis=0).reshape(-1)
    tweet_r = tweet.astype(jnp.int32).reshape(NW, bw)
    sent_r = sentiment.astype(jnp.int32).reshape(NW, bw)

    mesh = plsc.VectorSubcoreMesh(core_axis_name="c", subcore_axis_name="s")
    run = pl.kernel(
        functools.partial(_body, dim, tz, sz, bw),
        out_type=jax.ShapeDtypeStruct((NW, bw * 2 * dim), jnp.float32),
        mesh=mesh,
        scratch_types=[
            pltpu.VMEM((t_flat.shape[0],), jnp.float32),
            pltpu.VMEM((s_flat.shape[0],), jnp.float32),
            pltpu.VMEM((bw,), jnp.int32),
            pltpu.VMEM((bw,), jnp.int32),
            pltpu.VMEM((bw * 2 * dim,), jnp.float32),
            pltpu.SemaphoreType.DMA,
        ],
        compiler_params=pltpu.CompilerParams(needs_layout_passes=False),
    )
    out = run(t_flat, s_flat, tweet_r, sent_r)
    return out.reshape(b, 2 * dim)


# speedup vs baseline: 2.5400x; 1.2216x over previous
import functools

import jax
import jax.numpy as jnp
from jax import lax
from jax.experimental import pallas as pl
from jax.experimental.pallas import tpu as pltpu
from jax.experimental.pallas import tpu_sc as plsc

NC, NS, L = 2, 16, 16
NW = NC * NS


def _body(dim, tz, sz, bw,
          t_tab, s_tab, tweet, sent, out, ttab_v, stab_v, tidx, sidx,
          big, sem):
    wid = lax.axis_index("s") * NC + lax.axis_index("c")
    row0 = wid * bw
    cps = [
        pltpu.async_copy(t_tab, ttab_v.at[pl.ds(0, tz)], sem),
        pltpu.async_copy(s_tab, stab_v.at[pl.ds(0, sz)], sem),
        pltpu.async_copy(tweet.at[pl.ds(row0, bw)], tidx, sem),
        pltpu.async_copy(sent.at[pl.ds(row0, bw)], sidx, sem),
    ]
    for c in cps:
        c.wait()

    lanes = lax.iota(jnp.int32, L)
    zero = jnp.zeros((L,), jnp.float32)
    for k in range(dim // L):
        ttab_v[tz, pl.ds(k * L, L)] = zero
        stab_v[sz, pl.ds(k * L, L)] = zero

    def chunk(ch, _):
        base = pl.multiple_of(ch * L, L)
        t16 = tidx[pl.ds(base, L)]
        s16 = sidx[pl.ds(base, L)]
        m = t16 == 0
        tr = jnp.where(m, tz, t16)
        sr = jnp.where(m, sz, s16)
        rows = lanes + base
        for g in range(0, dim, 16):
            cols = [(lanes + (g + k)) & (dim - 1) for k in range(16)]
            tv = [plsc.load_gather(ttab_v, [tr, cl]) for cl in cols]
            for k, cl in enumerate(cols):
                plsc.store_scatter(big, [rows, cl], tv[k])
            sv = [plsc.load_gather(stab_v, [sr, cl]) for cl in cols]
            for k, cl in enumerate(cols):
                plsc.store_scatter(big, [rows, dim + cl], sv[k])
        return _

    lax.fori_loop(0, bw // L, chunk, None)
    pltpu.sync_copy(big, out.at[pl.ds(row0, bw)])


def kernel(tweet, sentiment, tweet_table, sentiment_table):
    b = tweet.shape[0]
    dim = tweet_table.shape[1]
    tz = tweet_table.shape[0]
    sz = sentiment_table.shape[0]
    bw = b // NW

    mesh = plsc.VectorSubcoreMesh(core_axis_name="c", subcore_axis_name="s")
    run = pl.kernel(
        functools.partial(_body, dim, tz, sz, bw),
        out_type=jax.ShapeDtypeStruct((b, 2 * dim), jnp.float32),
        mesh=mesh,
        scratch_types=[
            pltpu.VMEM((tz + 1, dim), jnp.float32),
            pltpu.VMEM((sz + 1, dim), jnp.float32),
            pltpu.VMEM((bw,), jnp.int32),
            pltpu.VMEM((bw,), jnp.int32),
            pltpu.VMEM((bw, 2 * dim), jnp.float32),
            pltpu.SemaphoreType.DMA,
        ],
        compiler_params=pltpu.CompilerParams(needs_layout_passes=False),
    )
    return run(tweet_table, sentiment_table,
               tweet.astype(jnp.int32), sentiment.astype(jnp.int32))


# speedup vs baseline: 3.0850x; 1.2146x over previous
import functools

import jax
import jax.numpy as jnp
from jax import lax
from jax.experimental import pallas as pl
from jax.experimental.pallas import tpu as pltpu
from jax.experimental.pallas import tpu_sc as plsc

NC, NS, L = 2, 16, 16
NW = NC * NS


def _body(dim, tz, sz, bw,
          t_tab, s_tab, tweet, sent, out, ttab_v, stab_v, tidx, sidx,
          big, sem):
    wid = lax.axis_index("s") * NC + lax.axis_index("c")
    row0 = wid * bw
    cps = [
        pltpu.async_copy(t_tab, ttab_v.at[pl.ds(0, tz)], sem),
        pltpu.async_copy(s_tab, stab_v.at[pl.ds(0, sz)], sem),
        pltpu.async_copy(tweet.at[pl.ds(row0, bw)], tidx, sem),
        pltpu.async_copy(sent.at[pl.ds(row0, bw)], sidx, sem),
    ]
    for c in cps:
        c.wait()

    lanes = lax.iota(jnp.int32, L)
    zero = jnp.zeros((L,), jnp.float32)
    for k in range(dim // L):
        ttab_v[tz, pl.ds(k * L, L)] = zero
        stab_v[sz, pl.ds(k * L, L)] = zero

    def chunk(ch, _):
        base = pl.multiple_of(ch * L, L)
        t16 = tidx[pl.ds(base, L)]
        s16 = sidx[pl.ds(base, L)]
        m = t16 == 0
        tr = jnp.where(m, tz, t16)
        sr = jnp.where(m, sz, s16)
        rows = lanes + base
        for g in range(0, dim, 16):
            cols = [(lanes + (g + k)) & (dim - 1) for k in range(16)]
            tv = [plsc.load_gather(ttab_v, [tr, cl]) for cl in cols]
            for k, cl in enumerate(cols):
                plsc.store_scatter(big, [cl, rows], tv[k])
            sv = [plsc.load_gather(stab_v, [sr, cl]) for cl in cols]
            for k, cl in enumerate(cols):
                plsc.store_scatter(big, [cl + dim, rows], sv[k])
        return _

    lax.fori_loop(0, bw // L, chunk, None)
    pltpu.sync_copy(big, out.at[:, pl.ds(row0, bw)])


def kernel(tweet, sentiment, tweet_table, sentiment_table):
    b = tweet.shape[0]
    dim = tweet_table.shape[1]
    tz = tweet_table.shape[0]
    sz = sentiment_table.shape[0]
    bw = b // NW

    mesh = plsc.VectorSubcoreMesh(core_axis_name="c", subcore_axis_name="s")
    run = pl.kernel(
        functools.partial(_body, dim, tz, sz, bw),
        out_type=jax.ShapeDtypeStruct((2 * dim, b), jnp.float32),
        mesh=mesh,
        scratch_types=[
            pltpu.VMEM((tz + 1, dim), jnp.float32),
            pltpu.VMEM((sz + 1, dim), jnp.float32),
            pltpu.VMEM((bw,), jnp.int32),
            pltpu.VMEM((bw,), jnp.int32),
            pltpu.VMEM((2 * dim, bw), jnp.float32),
            pltpu.SemaphoreType.DMA,
        ],
        compiler_params=pltpu.CompilerParams(needs_layout_passes=False),
    )
    out_t = run(tweet_table, sentiment_table,
                tweet.astype(jnp.int32), sentiment.astype(jnp.int32))
    return out_t.T


# speedup vs baseline: 3.4071x; 1.1044x over previous
import functools

import jax
import jax.numpy as jnp
from jax import lax
from jax.experimental import pallas as pl
from jax.experimental.pallas import tpu as pltpu
from jax.experimental.pallas import tpu_sc as plsc

NC, NS, L = 2, 16, 16
NW = NC * NS
SREP = 4


def _body(dim, tz, sz, bw,
          t_tab, s_tab, tweet, sent, out, ttab_v, stab_v, stab2_v, tidx,
          sidx, big, sem):
    wid = lax.axis_index("s") * NC + lax.axis_index("c")
    row0 = wid * bw
    cps = [
        pltpu.async_copy(t_tab, ttab_v, sem),
        pltpu.async_copy(s_tab, stab_v, sem),
        pltpu.async_copy(tweet.at[pl.ds(row0, bw)], tidx, sem),
        pltpu.async_copy(sent.at[pl.ds(row0, bw)], sidx, sem),
    ]
    for c in cps:
        c.wait()

    lanes = lax.iota(jnp.int32, L)
    zero = jnp.zeros((L,), jnp.float32)
    ncol = (sz + 4) * SREP

    for r in range(dim):
        for k in range(ncol // L):
            stab2_v[r, pl.ds(k * L, L)] = zero
    for s in range(sz):
        for k in range(dim // L):
            v = stab_v[s, pl.ds(k * L, L)]
            for rep in range(SREP):
                plsc.store_scatter(
                    stab2_v,
                    [k * L + lanes, jnp.full((L,), s + 8 * rep, jnp.int32)], v)

    rep_off = (lanes & (SREP - 1)) * 8

    def chunk(ch, _):
        base = pl.multiple_of(ch * L, L)
        t16 = tidx[pl.ds(base, L)]
        s16 = sidx[pl.ds(base, L)]
        m = t16 == 0
        sr = jnp.where(m, sz, s16) + rep_off
        for g in range(0, 2 * dim, L):
            vals = []
            for k in range(L):
                c = g + k
                if c < dim:
                    v = plsc.load_gather(
                        ttab_v, [jnp.full((L,), c, jnp.int32), t16])
                    vals.append(jnp.where(m, 0.0, v))
                else:
                    vals.append(plsc.load_gather(
                        stab2_v, [jnp.full((L,), c - dim, jnp.int32), sr]))
            for k in range(L):
                big[g + k, pl.ds(base, L)] = vals[k]
        return _

    half = bw // 2
    lax.fori_loop(0, half // L, chunk, None)
    cp1 = pltpu.async_copy(big.at[:, pl.ds(0, half)],
                           out.at[:, pl.ds(row0, half)], sem)
    lax.fori_loop(half // L, bw // L, chunk, None)
    cp1.wait()
    pltpu.sync_copy(big.at[:, pl.ds(half, half)],
                    out.at[:, pl.ds(row0 + half, half)])


def kernel(tweet, sentiment, tweet_table, sentiment_table):
    b = tweet.shape[0]
    dim = tweet_table.shape[1]
    tz = tweet_table.shape[0]
    sz = sentiment_table.shape[0]
    bw = b // NW

    mesh = plsc.VectorSubcoreMesh(core_axis_name="c", subcore_axis_name="s")
    run = pl.kernel(
        functools.partial(_body, dim, tz, sz, bw),
        out_type=jax.ShapeDtypeStruct((2 * dim, b), jnp.float32),
        mesh=mesh,
        scratch_types=[
            pltpu.VMEM((dim, tz), jnp.float32),
            pltpu.VMEM((sz, dim), jnp.float32),
            pltpu.VMEM((dim, (sz + 4) * SREP), jnp.float32),
            pltpu.VMEM((bw,), jnp.int32),
            pltpu.VMEM((bw,), jnp.int32),
            pltpu.VMEM((2 * dim, bw), jnp.float32),
            pltpu.SemaphoreType.DMA,
        ],
        compiler_params=pltpu.CompilerParams(needs_layout_passes=False),
    )
    out_t = run(tweet_table.T, sentiment_table,
                tweet.astype(jnp.int32), sentiment.astype(jnp.int32))
    return out_t.T


# speedup vs baseline: 3.4640x; 1.0167x over previous
import functools

import jax
import jax.numpy as jnp
from jax import lax
from jax.experimental import pallas as pl
from jax.experimental.pallas import tpu as pltpu
from jax.experimental.pallas import tpu_sc as plsc

NC, NS, L = 2, 16, 16
NW = NC * NS
SREP = 4


def _body(dim, tz, sz, bw,
          t_tab, s_tab, tweet, sent, out, ttab_v, stab_v, stab2_v, tidx,
          sidx, big, sem):
    wid = lax.axis_index("s") * NC + lax.axis_index("c")
    row0 = wid * bw
    cps = [
        pltpu.async_copy(t_tab, ttab_v, sem),
        pltpu.async_copy(s_tab, stab_v, sem),
        pltpu.async_copy(tweet.at[pl.ds(row0, bw)], tidx, sem),
        pltpu.async_copy(sent.at[pl.ds(row0, bw)], sidx, sem),
    ]
    for c in cps:
        c.wait()

    lanes = lax.iota(jnp.int32, L)
    zero = jnp.zeros((L,), jnp.float32)
    ncol = (sz + 4) * SREP

    for r in range(dim):
        for k in range(ncol // L):
            stab2_v[r, pl.ds(k * L, L)] = zero
    for s in range(sz):
        for k in range(dim // L):
            v = stab_v[s, pl.ds(k * L, L)]
            for rep in range(SREP):
                plsc.store_scatter(
                    stab2_v,
                    [k * L + lanes, jnp.full((L,), s + 8 * rep, jnp.int32)], v)

    rep_off = (lanes & (SREP - 1)) * 8

    def chunk_body(ch):
        base = pl.multiple_of(ch * L, L)
        t16 = tidx[pl.ds(base, L)]
        s16 = sidx[pl.ds(base, L)]
        m = t16 == 0
        sr = jnp.where(m, sz, s16) + rep_off
        for g in range(0, 2 * dim, L):
            vals = []
            for k in range(L):
                c = g + k
                if c < dim:
                    v = plsc.load_gather(
                        ttab_v, [jnp.full((L,), c, jnp.int32), t16])
                    vals.append(jnp.where(m, 0.0, v))
                else:
                    vals.append(plsc.load_gather(
                        stab2_v, [jnp.full((L,), c - dim, jnp.int32), sr]))
            for k in range(L):
                big[g + k, pl.ds(base, L)] = vals[k]

    half = bw // 2

    @plsc.parallel_loop(0, half // L)
    def _loop1(ch):
        chunk_body(ch)

    cp1 = pltpu.async_copy(big.at[:, pl.ds(0, half)],
                           out.at[:, pl.ds(row0, half)], sem)

    @plsc.parallel_loop(half // L, bw // L)
    def _loop2(ch):
        chunk_body(ch)

    cp1.wait()
    pltpu.sync_copy(big.at[:, pl.ds(half, half)],
                    out.at[:, pl.ds(row0 + half, half)])


def kernel(tweet, sentiment, tweet_table, sentiment_table):
    b = tweet.shape[0]
    dim = tweet_table.shape[1]
    tz = tweet_table.shape[0]
    sz = sentiment_table.shape[0]
    bw = b // NW

    mesh = plsc.VectorSubcoreMesh(core_axis_name="c", subcore_axis_name="s")
    run = pl.kernel(
        functools.partial(_body, dim, tz, sz, bw),
        out_type=jax.ShapeDtypeStruct((2 * dim, b), jnp.float32),
        mesh=mesh,
        scratch_types=[
            pltpu.VMEM((dim, tz), jnp.float32),
            pltpu.VMEM((sz, dim), jnp.float32),
            pltpu.VMEM((dim, (sz + 4) * SREP), jnp.float32),
            pltpu.VMEM((bw,), jnp.int32),
            pltpu.VMEM((bw,), jnp.int32),
            pltpu.VMEM((2 * dim, bw), jnp.float32),
            pltpu.SemaphoreType.DMA,
        ],
        compiler_params=pltpu.CompilerParams(needs_layout_passes=False),
    )
    out_t = run(tweet_table.T, sentiment_table,
                tweet.astype(jnp.int32), sentiment.astype(jnp.int32))
    return out_t.T
